# baseline (speedup 1.0000x reference)
import functools

import jax
import jax.numpy as jnp
from jax import lax
from jax.experimental import pallas as pl
from jax.experimental.pallas import tpu as pltpu
from jax.experimental.pallas import tpu_sc as plsc

_D = 16
_NC = 2
_NS = 16
_NW = _NC * _NS


@functools.lru_cache(maxsize=None)
def _make_gather(batch: int, toks: int, chunk: int):
  rows_per_w = batch // _NW
  nchunks = rows_per_w // chunk
  assert rows_per_w % chunk == 0 and nchunks % 2 == 0
  mesh = plsc.VectorSubcoreMesh(core_axis_name="c", subcore_axis_name="s")

  @functools.partial(
      pl.kernel,
      out_type=jax.ShapeDtypeStruct((batch, toks, _D), jnp.float32),
      mesh=mesh,
      scratch_types=[
          pltpu.VMEM((rows_per_w, toks), jnp.int32),
          pltpu.VMEM((2, chunk, toks, _D), jnp.float32),
          pltpu.SemaphoreType.DMA,
          pltpu.SemaphoreType.DMA,
          pltpu.SemaphoreType.DMA,
          pltpu.SemaphoreType.DMA,
      ],
      compiler_params=pltpu.CompilerParams(use_tc_tiling_on_sc=False,
                                           needs_layout_passes=True),
  )
  def gather_kernel(idx_hbm, table_hbm, out_hbm, idx_v, rows_v,
                    gsem0, gsem1, osem0, osem1):
    wid = lax.axis_index("s") * _NC + lax.axis_index("c")
    base = wid * rows_per_w
    gsems = (gsem0, gsem1)
    osems = (osem0, osem1)

    pltpu.sync_copy(idx_hbm.at[pl.ds(base, rows_per_w)], idx_v)

    def chunk_body(i, buf):
      row0 = i * chunk
      copies = []
      for j in range(chunk):
        copies.append(pltpu.async_copy(
            table_hbm.at[idx_v.at[row0 + j]],
            rows_v.at[buf].at[j], gsems[buf]))
      for c in copies:
        c.wait()
      return pltpu.async_copy(
          rows_v.at[buf], out_hbm.at[pl.ds(base + row0, chunk)], osems[buf])

    def pair_body(p, carry):
      i = p * 2
      s0 = chunk_body(i, 0)
      s1 = chunk_body(i + 1, 1)
      s0.wait()
      s1.wait()
      return carry

    lax.fori_loop(0, nchunks // 2, pair_body, 0)

  return gather_kernel


def kernel(indices, table):
  batch, toks = indices.shape
  return _make_gather(batch, toks, 32)(indices, table)


# speedup vs baseline: 1.0816x; 1.0816x over previous
import functools

import jax
import jax.numpy as jnp
from jax import lax
from jax.experimental import pallas as pl
from jax.experimental.pallas import tpu as pltpu
from jax.experimental.pallas import tpu_sc as plsc

_D = 16
_NC = 2
_NS = 16
_NW = _NC * _NS


@functools.lru_cache(maxsize=None)
def _make_gather(batch: int, toks: int, chunk: int):
  rows_per_w = batch // _NW
  nchunks = rows_per_w // chunk
  assert rows_per_w % chunk == 0 and nchunks % 2 == 0
  mesh = plsc.VectorSubcoreMesh(core_axis_name="c", subcore_axis_name="s")

  @functools.partial(
      pl.kernel,
      out_type=jax.ShapeDtypeStruct((batch, toks, _D), jnp.float32),
      mesh=mesh,
      scratch_types=[
          pltpu.VMEM((rows_per_w, toks), jnp.int32),
          pltpu.VMEM((2, chunk, toks, _D), jnp.float32),
          pltpu.SemaphoreType.DMA,
          pltpu.SemaphoreType.DMA,
          pltpu.SemaphoreType.DMA,
          pltpu.SemaphoreType.DMA,
      ],
      compiler_params=pltpu.CompilerParams(use_tc_tiling_on_sc=False,
                                           needs_layout_passes=True),
  )
  def gather_kernel(idx_hbm, table_hbm, out_hbm, idx_v, rows_v,
                    gsem0, gsem1, osem0, osem1):
    wid = lax.axis_index("s") * _NC + lax.axis_index("c")
    base = wid * rows_per_w
    gsems = (gsem0, gsem1)
    osems = (osem0, osem1)

    pltpu.sync_copy(idx_hbm.at[pl.ds(base, rows_per_w)], idx_v)

    def chunk_body(i, buf):
      row0 = i * chunk
      copies = []
      for j in range(chunk):
        copies.append(pltpu.async_copy(
            table_hbm.at[idx_v.at[row0 + j]],
            rows_v.at[buf].at[j], gsems[buf]))
      for c in copies:
        c.wait()
      return pltpu.async_copy(
          rows_v.at[buf], out_hbm.at[pl.ds(base + row0, chunk)], osems[buf])

    def pair_body(p, carry):
      i = p * 2
      s0 = chunk_body(i, 0)
      s1 = chunk_body(i + 1, 1)
      s0.wait()
      s1.wait()
      return carry

    lax.fori_loop(0, nchunks // 2, pair_body, 0)

  return gather_kernel


def kernel(indices, table):
  batch, toks = indices.shape
  half = batch // 2
  g = _make_gather(half, toks, 64)
  o1 = g(indices[:half], table)
  o2 = g(indices[half:], table)
  return jnp.concatenate([o1, o2], axis=0)


# speedup vs baseline: 1.1431x; 1.0569x over previous
import functools

import jax
import jax.numpy as jnp
from jax import lax
from jax.experimental import pallas as pl
from jax.experimental.pallas import tpu as pltpu
from jax.experimental.pallas import tpu_sc as plsc

_D = 16
_NC = 2
_NS = 16
_NW = _NC * _NS


@functools.lru_cache(maxsize=None)
def _make_gather(batch: int, toks: int, chunk: int):
  rows_per_w = batch // _NW
  nchunks = rows_per_w // chunk
  assert rows_per_w % chunk == 0 and nchunks % 2 == 0
  mesh = plsc.VectorSubcoreMesh(core_axis_name="c", subcore_axis_name="s")

  @functools.partial(
      pl.kernel,
      out_type=jax.ShapeDtypeStruct((batch, toks, _D), jnp.float32),
      mesh=mesh,
      scratch_types=[
          pltpu.VMEM((rows_per_w, toks), jnp.int32),
          pltpu.VMEM((2, chunk, toks, _D), jnp.float32),
          pltpu.SemaphoreType.DMA,
          pltpu.SemaphoreType.DMA,
          pltpu.SemaphoreType.DMA,
          pltpu.SemaphoreType.DMA,
      ],
      compiler_params=pltpu.CompilerParams(use_tc_tiling_on_sc=False,
                                           needs_layout_passes=True),
  )
  def gather_kernel(idx_hbm, table_hbm, out_hbm, idx_v, rows_v,
                    gsem0, gsem1, osem0, osem1):
    wid = lax.axis_index("s") * _NC + lax.axis_index("c")
    base = wid * rows_per_w
    gsems = (gsem0, gsem1)
    osems = (osem0, osem1)

    pltpu.sync_copy(idx_hbm.at[pl.ds(base, rows_per_w)], idx_v)

    def chunk_body(i, buf):
      row0 = i * chunk
      copies = []
      for j in range(chunk):
        copies.append(pltpu.async_copy(
            table_hbm.at[idx_v.at[row0 + j]],
            rows_v.at[buf].at[j], gsems[buf]))
      for c in copies:
        c.wait()
      return pltpu.async_copy(
          rows_v.at[buf], out_hbm.at[pl.ds(base + row0, chunk)], osems[buf])

    def pair_body(p, carry):
      i = p * 2
      s0 = chunk_body(i, 0)
      s1 = chunk_body(i + 1, 1)
      s0.wait()
      s1.wait()
      return carry

    lax.fori_loop(0, nchunks // 2, pair_body, 0)

  return gather_kernel


def kernel(indices, table):
  batch, toks = indices.shape
  nsplit = 4
  part = batch // nsplit
  g = _make_gather(part, toks, 64)
  outs = [g(indices[i * part:(i + 1) * part], table) for i in range(nsplit)]
  return jnp.concatenate(outs, axis=0)
